# baseline (speedup 1.0000x reference)
import jax
import jax.numpy as jnp
from jax import lax
from jax.experimental import pallas as pl
from jax.experimental.pallas import tpu as pltpu
from jax.experimental.pallas import tpu_sc as plsc

BATCH = 16384
SEQ_LEN = 50
DIM = 32
N_EMB = 1000000

_TPW = 2048
_TPGRID = 64
_NQ = 1 << 17
_N8 = _TPGRID * _TPW

_NC = 2
_NS = 16
_NW = _NC * _NS
_ROWS_PER_W = BATCH // _NW
_BLK_ROWS = 32
_BLK_IDX = _BLK_ROWS * SEQ_LEN
_NBLK = _ROWS_PER_W // _BLK_ROWS
_NBUF = 2
_NGRP = _NBLK // _NBUF
_INV_L = float(1.0 / SEQ_LEN)
_HI_MASK = -65536


def _tp_kernel(*refs):
    xs, o_ref = refs[:8], refs[8]
    packs = []
    for x in xs:
        bits = lax.bitcast_convert_type(x[...], jnp.int32)
        lo = lax.shift_right_logical(bits[0:16, :], 16)
        hi = bits[16:32, :] & _HI_MASK
        packs.append(hi | lo)
    stacked = jnp.concatenate(packs, axis=0)
    o_ref[...] = jnp.transpose(stacked)


def _tc_relayout(table_t):
    in_specs = [
        pl.BlockSpec(
            (DIM, _TPW),
            lambda j, g=g: (0, g * 64 + j if g < 7
                            else 7 * 64 + jnp.minimum(j, 40)),
        )
        for g in range(8)
    ]
    return pl.pallas_call(
        _tp_kernel,
        out_shape=jax.ShapeDtypeStruct((_N8, 128), jnp.int32),
        grid=(_TPGRID,),
        in_specs=in_specs,
        out_specs=pl.BlockSpec((_TPW, 128), lambda j: (j, 0)),
    )(*([table_t] * 8))


def _sc_kernel(x_hbm, table_hbm, out_hbm, x2d_v, idx_v, rows_v, out_v, *sems):
    wid = lax.axis_index("s") * _NC + lax.axis_index("c")
    base_col = wid * _ROWS_PER_W

    pltpu.sync_copy(x_hbm.at[:, pl.ds(base_col, _ROWS_PER_W)], x2d_v)

    lane50 = lax.iota(jnp.int32, 16) * 50

    def shuffle(c, _):
        for j in range(SEQ_LEN):
            v = x2d_v[j, pl.ds(c * 16, 16)]
            g = ((v & (_NQ - 1)) << 3) | lax.shift_right_logical(v, 17)
            plsc.store_scatter(idx_v, [lane50 + (c * 800 + j)], g)
        return ()

    lax.fori_loop(0, _ROWS_PER_W // 16, shuffle, ())

    for b in range(_NBUF):
        pltpu.async_copy(
            table_hbm.at[idx_v.at[pl.ds(b * _BLK_IDX, _BLK_IDX)]],
            rows_v.at[b], sems[b])

    lane_d = lax.iota(jnp.int32, 16)

    def reduce_blk(b, blk):
        def body(r, _):
            acc = [jnp.zeros((16,), jnp.float32) for _ in range(4)]
            for j in range(SEQ_LEN):
                v = rows_v[b, r * SEQ_LEN + j, :]
                k = (j & 1) << 1
                acc[k] = acc[k] + plsc.bitcast(v << 16, jnp.float32)
                acc[k + 1] = acc[k + 1] + plsc.bitcast(v, jnp.float32)
            col = blk * _BLK_ROWS + r
            plsc.store_scatter(
                out_v, [lane_d, lane_d * 0 + col],
                (acc[0] + acc[2]) * _INV_L)
            plsc.store_scatter(
                out_v, [lane_d + 16, lane_d * 0 + col],
                (acc[1] + acc[3]) * _INV_L)
            return ()

        lax.fori_loop(0, _BLK_ROWS, body, ())

    def group(g, _):
        for b in range(_NBUF):
            blk = g * _NBUF + b
            pltpu.make_async_copy(
                table_hbm.at[idx_v.at[pl.ds(blk * _BLK_IDX, _BLK_IDX)]],
                rows_v.at[b], sems[b]
            ).wait()
            reduce_blk(b, blk)

            @pl.when(g < _NGRP - 1)
            def _():
                pltpu.async_copy(
                    table_hbm.at[
                        idx_v.at[pl.ds((blk + _NBUF) * _BLK_IDX, _BLK_IDX)]],
                    rows_v.at[b], sems[b])

        return ()

    lax.fori_loop(0, _NGRP, group, ())

    pltpu.sync_copy(out_v, out_hbm.at[:, pl.ds(base_col, _ROWS_PER_W)])


@jax.jit
def kernel(x, table):
    tbl_pack = _tc_relayout(table.T)
    table_rm = tbl_pack.reshape(_N8 * 8, 16)
    mesh = plsc.VectorSubcoreMesh(
        core_axis_name="c", subcore_axis_name="s",
        num_cores=_NC, num_subcores=_NS,
    )
    run = pl.kernel(
        _sc_kernel,
        out_type=jax.ShapeDtypeStruct((DIM, BATCH), jnp.float32),
        mesh=mesh,
        scratch_types=[
            pltpu.VMEM((SEQ_LEN, _ROWS_PER_W), jnp.int32),
            pltpu.VMEM((_NW_IDX := _ROWS_PER_W * SEQ_LEN,), jnp.int32),
            pltpu.VMEM((_NBUF, _BLK_IDX, 16), jnp.int32),
            pltpu.VMEM((DIM, _ROWS_PER_W), jnp.float32),
        ] + [pltpu.SemaphoreType.DMA] * _NBUF,
        compiler_params=pltpu.CompilerParams(
            use_tc_tiling_on_sc=False, needs_layout_passes=False),
    )
    return run(x.T, table_rm).T
